# baseline (speedup 1.0000x reference)
import functools

import jax
import jax.numpy as jnp
from jax import lax
from jax.experimental import pallas as pl
from jax.experimental.pallas import tpu as pltpu
from jax.experimental.pallas import tpu_sc as plsc

B = 16
N = 2048
D = 1024
NC = 2
NS = 16
NW = NC * NS
DW = D // NW
R = 256
U = 8
NEG = -99999.0


def _body(hidden, counts, pooled, indices, counts_v, buf, outv, outi):
    c = lax.axis_index("c")
    s = lax.axis_index("s")
    w = s * NC + c
    off = w * DW

    pltpu.sync_copy(counts, counts_v)
    lane = lax.broadcasted_iota(jnp.int32, (16,), 0)
    cnt = counts_v[...]
    eff = jnp.minimum(jnp.maximum(cnt, 1), N)

    col1 = off + lane
    col2 = off + 16 + lane

    for b in range(B):
        effb = jnp.max(jnp.where(lane == b, eff, 0))
        nch = (effb + (R - 1)) // R
        effb_v = jnp.full((16,), effb, jnp.int32)

        def chunk_body(ci, carry, b=b, effb_v=effb_v):
            m1, m2, i1, i2, g = carry
            base = ci * R
            pltpu.sync_copy(hidden.at[b, pl.ds(base, R), pl.ds(off, DW)], buf)

            def rows_body(k, carry):
                m1, m2, i1, i2, g = carry
                for u in range(U):
                    r = k * U + u
                    x1 = buf[r, 0:16]
                    x2 = buf[r, 16:32]
                    valid = g < effb_v
                    c1 = (x1 > m1) & valid
                    c2 = (x2 > m2) & valid
                    m1 = jnp.where(c1, x1, m1)
                    m2 = jnp.where(c2, x2, m2)
                    i1 = jnp.where(c1, g, i1)
                    i2 = jnp.where(c2, g, i2)
                    g = g + 1
                return m1, m2, i1, i2, g

            return lax.fori_loop(0, R // U, rows_body, (m1, m2, i1, i2, g))

        zi = jnp.zeros((16,), jnp.int32)
        init = (jnp.full((16,), NEG, jnp.float32),
                jnp.full((16,), NEG, jnp.float32), zi, zi, zi)
        m1, m2, i1, i2, _ = lax.fori_loop(0, nch, chunk_body, init)

        outv[0:16] = m1
        outv[16:32] = m2
        outi[0:16] = i1 * D + col1
        outi[16:32] = i2 * D + col2
        pltpu.sync_copy(outv, pooled.at[b, pl.ds(off, DW)])
        pltpu.sync_copy(outi, indices.at[b, pl.ds(off, DW)])


_mesh = plsc.VectorSubcoreMesh(core_axis_name="c", subcore_axis_name="s")

_sc_call = pl.kernel(
    _body,
    out_type=(
        jax.ShapeDtypeStruct((B, D), jnp.float32),
        jax.ShapeDtypeStruct((B, D), jnp.int32),
    ),
    mesh=_mesh,
    scratch_types=[
        pltpu.VMEM((16,), jnp.int32),
        pltpu.VMEM((R, DW), jnp.float32),
        pltpu.VMEM((DW,), jnp.float32),
        pltpu.VMEM((DW,), jnp.int32),
    ],
    compiler_params=pltpu.CompilerParams(
        use_tc_tiling_on_sc=False, needs_layout_passes=False),
)


@jax.jit
def kernel(hidden, child_counts):
    pooled, idx = _sc_call(hidden, child_counts)
    return pooled.reshape(B, 1, D), idx.reshape(B, 1, 1, D)


# speedup vs baseline: 1.3693x; 1.3693x over previous
import jax
import jax.numpy as jnp
from jax import lax
from jax.experimental import pallas as pl
from jax.experimental.pallas import tpu as pltpu
from jax.experimental.pallas import tpu_sc as plsc

B = 16
N = 2048
D = 1024
NC = 2
NS = 16
BL = B // NC
RB = 48
NV = D // 16
FB = 8
CW = D // NS
NEG = -99999.0


def _body(hidden, counts, pooled, indices,
          counts_v, buf0, buf1, acc_v, acc_i, shv, shi,
          cmb_v, cmb_i, outv, outi, sem0, sem1):
    c = lax.axis_index("c")
    s = lax.axis_index("s")

    pltpu.sync_copy(counts, counts_v)
    lane = lax.broadcasted_iota(jnp.int32, (16,), 0)
    eff_all = jnp.minimum(jnp.maximum(counts_v[...], 1), N)

    negv = jnp.full((16,), NEG, jnp.float32)
    zerov = jnp.zeros((16,), jnp.int32)

    def init_b(bl, _):
        for v in range(NV):
            acc_v[bl, pl.ds(v * 16, 16)] = negv
            acc_i[bl, pl.ds(v * 16, 16)] = zerov
        return 0
    lax.fori_loop(0, BL, init_b, 0)

    effs, nchs = [], []
    for t in range(BL):
        e = jnp.max(jnp.where(lane == c * BL + t, eff_all, 0))
        effs.append(e)
        nchs.append((e + (RB - 1)) // RB)
    pref = [jnp.int32(0)]
    for t in range(BL):
        pref.append(pref[t] + nchs[t])
    total = pref[BL]
    mine = jnp.maximum((total - s + (NS - 1)) // NS, 0)

    def locate(j):
        b_l = jnp.int32(0)
        base = jnp.int32(0)
        e_s = effs[0]
        for t in range(1, BL):
            cond = j >= pref[t]
            b_l = jnp.where(cond, t, b_l)
            base = jnp.where(cond, pref[t], base)
            e_s = jnp.where(cond, effs[t], e_s)
        return b_l, j - base, e_s

    def dma(b_l, k, buf, sem):
        return pltpu.make_async_copy(
            hidden.at[c * BL + b_l, pl.ds(k * RB, RB), :], buf, sem)

    def compute_item(b_l, k, e_s, buf):
        cnt = jnp.minimum(RB, e_s - k * RB)
        for fb in range(FB):
            m = [acc_v[b_l, pl.ds(fb * 128 + v * 16, 16)] for v in range(8)]
            ii = [acc_i[b_l, pl.ds(fb * 128 + v * 16, 16)] for v in range(8)]
            g = jnp.full((16,), k * RB, jnp.int32)

            def row_body(r, carry):
                mm = list(carry[0:8])
                jj = list(carry[8:16])
                gg = carry[16]
                for v in range(8):
                    x = buf[r, pl.ds(fb * 128 + v * 16, 16)]
                    cge = x > mm[v]
                    mm[v] = jnp.where(cge, x, mm[v])
                    jj[v] = jnp.where(cge, gg, jj[v])
                return tuple(mm) + tuple(jj) + (gg + 1,)

            out = lax.fori_loop(0, cnt, row_body, tuple(m) + tuple(ii) + (g,))
            for v in range(8):
                acc_v[b_l, pl.ds(fb * 128 + v * 16, 16)] = out[v]
                acc_i[b_l, pl.ds(fb * 128 + v * 16, 16)] = out[8 + v]

    b0, k0, e0 = locate(s)

    @pl.when(mine > 0)
    def _():
        dma(b0, k0, buf0, sem0).start()

    bufs = (buf0, buf1)
    sems = (sem0, sem1)

    def pair_body(p, _):
        for q in (0, 1):
            item = 2 * p + q

            @pl.when(item < mine)
            def _(item=item, q=q):
                j = s + NS * item
                b_l, k, e_s = locate(j)
                dma(b_l, k, bufs[q], sems[q]).wait()
                nitem = item + 1

                @pl.when(nitem < mine)
                def _():
                    nb, nk, _ne = locate(s + NS * nitem)
                    dma(nb, nk, bufs[1 - q], sems[1 - q]).start()

                compute_item(b_l, k, e_s, bufs[q])
        return 0

    lax.fori_loop(0, (mine + 1) // 2, pair_body, 0)

    def pub_b(bl, _):
        pltpu.sync_copy(acc_v.at[bl], shv.at[c, s, bl])
        pltpu.sync_copy(acc_i.at[bl], shi.at[c, s, bl])
        return 0
    lax.fori_loop(0, BL, pub_b, 0)
    plsc.subcore_barrier()

    def comb_b(bl, _):
        bg = c * BL + bl
        pltpu.sync_copy(shv.at[c, :, bl, pl.ds(CW * s, CW)], cmb_v)
        pltpu.sync_copy(shi.at[c, :, bl, pl.ds(CW * s, CW)], cmb_i)
        for t in range(CW // 16):
            m = cmb_v[0, pl.ds(t * 16, 16)]
            ii = cmb_i[0, pl.ds(t * 16, 16)]
            for j in range(1, NS):
                x = cmb_v[j, pl.ds(t * 16, 16)]
                ix = cmb_i[j, pl.ds(t * 16, 16)]
                cge = x > m
                m = jnp.where(cge, x, m)
                ii = jnp.where(cge, ix, ii)
            col = CW * s + t * 16 + lane
            outv[pl.ds(t * 16, 16)] = m
            outi[pl.ds(t * 16, 16)] = ii * D + col
        pltpu.sync_copy(outv, pooled.at[bg, pl.ds(CW * s, CW)])
        pltpu.sync_copy(outi, indices.at[bg, pl.ds(CW * s, CW)])
        return 0
    lax.fori_loop(0, BL, comb_b, 0)


_mesh = plsc.VectorSubcoreMesh(core_axis_name="c", subcore_axis_name="s")

_sc_call = pl.kernel(
    _body,
    out_type=(
        jax.ShapeDtypeStruct((B, D), jnp.float32),
        jax.ShapeDtypeStruct((B, D), jnp.int32),
    ),
    mesh=_mesh,
    scratch_types=[
        pltpu.VMEM((16,), jnp.int32),
        pltpu.VMEM((RB, D), jnp.float32),
        pltpu.VMEM((RB, D), jnp.float32),
        pltpu.VMEM((BL, D), jnp.float32),
        pltpu.VMEM((BL, D), jnp.int32),
        pltpu.HBM((NC, NS, BL, D), jnp.float32),
        pltpu.HBM((NC, NS, BL, D), jnp.int32),
        pltpu.VMEM((NS, CW), jnp.float32),
        pltpu.VMEM((NS, CW), jnp.int32),
        pltpu.VMEM((CW,), jnp.float32),
        pltpu.VMEM((CW,), jnp.int32),
        pltpu.SemaphoreType.DMA,
        pltpu.SemaphoreType.DMA,
    ],
    compiler_params=pltpu.CompilerParams(
        use_tc_tiling_on_sc=False, needs_layout_passes=False),
)


@jax.jit
def kernel(hidden, child_counts):
    pooled, idx = _sc_call(hidden, child_counts)
    return pooled.reshape(B, 1, D), idx.reshape(B, 1, 1, D)


# speedup vs baseline: 3.3072x; 2.4153x over previous
import jax
import jax.numpy as jnp
from jax import lax
from jax.experimental import pallas as pl
from jax.experimental.pallas import tpu as pltpu
from jax.experimental.pallas import tpu_sc as plsc

B = 16
N = 2048
D = 1024
NC = 2
NS = 16
BL = B // NC
RB = 48
NV = D // 16
FB = 8
NEG = -99999.0


def _sc_body(hidden, counts, pv, pi, counts_v, buf0, buf1, acc_v, acc_i,
             sem0, sem1):
    c = lax.axis_index("c")
    s = lax.axis_index("s")

    pltpu.sync_copy(counts, counts_v)
    lane = lax.broadcasted_iota(jnp.int32, (16,), 0)
    eff_all = jnp.minimum(jnp.maximum(counts_v[...], 1), N)

    negv = jnp.full((16,), NEG, jnp.float32)
    zerov = jnp.zeros((16,), jnp.int32)

    def init_b(bl, _):
        for v in range(NV):
            acc_v[pl.ds(bl * D + v * 16, 16)] = negv
            acc_i[pl.ds(bl * D + v * 16, 16)] = zerov
        return 0
    lax.fori_loop(0, BL, init_b, 0)

    effs, nchs = [], []
    for t in range(BL):
        e = jnp.max(jnp.where(lane == c * BL + t, eff_all, 0))
        effs.append(e)
        nchs.append((e + (RB - 1)) // RB)
    pref = [jnp.int32(0)]
    for t in range(BL):
        pref.append(pref[t] + nchs[t])
    total = pref[BL]
    mine = jnp.maximum((total - s + (NS - 1)) // NS, 0)

    def locate(j):
        b_l = jnp.int32(0)
        base = jnp.int32(0)
        e_s = effs[0]
        for t in range(1, BL):
            cond = j >= pref[t]
            b_l = jnp.where(cond, t, b_l)
            base = jnp.where(cond, pref[t], base)
            e_s = jnp.where(cond, effs[t], e_s)
        return b_l, j - base, e_s

    def dma(b_l, k, buf, sem):
        return pltpu.make_async_copy(
            hidden.at[c * BL + b_l, pl.ds(k * RB, RB), :], buf, sem)

    def compute_item(b_l, k, e_s, buf):
        cnt = jnp.minimum(RB, e_s - k * RB)
        for fb in range(FB):
            m = [acc_v[pl.ds(b_l * D + fb * 128 + v * 16, 16)]
                 for v in range(8)]
            ii = [acc_i[pl.ds(b_l * D + fb * 128 + v * 16, 16)]
                  for v in range(8)]
            g = jnp.full((16,), k * RB, jnp.int32)

            def row_body(r, carry):
                mm = list(carry[0:8])
                jj = list(carry[8:16])
                gg = carry[16]
                for v in range(8):
                    x = buf[r, pl.ds(fb * 128 + v * 16, 16)]
                    cge = x > mm[v]
                    mm[v] = jnp.where(cge, x, mm[v])
                    jj[v] = jnp.where(cge, gg, jj[v])
                return tuple(mm) + tuple(jj) + (gg + 1,)

            out = lax.fori_loop(0, cnt, row_body, tuple(m) + tuple(ii) + (g,))
            for v in range(8):
                acc_v[pl.ds(b_l * D + fb * 128 + v * 16, 16)] = out[v]
                acc_i[pl.ds(b_l * D + fb * 128 + v * 16, 16)] = out[8 + v]

    b0, k0, e0 = locate(s)

    @pl.when(mine > 0)
    def _():
        dma(b0, k0, buf0, sem0).start()

    bufs = (buf0, buf1)
    sems = (sem0, sem1)

    def pair_body(p, _):
        for q in (0, 1):
            item = 2 * p + q

            @pl.when(item < mine)
            def _(item=item, q=q):
                j = s + NS * item
                b_l, k, e_s = locate(j)
                dma(b_l, k, bufs[q], sems[q]).wait()
                nitem = item + 1

                @pl.when(nitem < mine)
                def _():
                    nb, nk, _ne = locate(s + NS * nitem)
                    dma(nb, nk, bufs[1 - q], sems[1 - q]).start()

                compute_item(b_l, k, e_s, bufs[q])
        return 0

    lax.fori_loop(0, (mine + 1) // 2, pair_body, 0)

    w = c * NS + s
    for bl in range(BL):
        pltpu.sync_copy(acc_v.at[pl.ds(bl * D, D)],
                        pv.at[pl.ds((w * BL + bl) * D, D)])
        pltpu.sync_copy(acc_i.at[pl.ds(bl * D, D)],
                        pi.at[pl.ds((w * BL + bl) * D, D)])


_mesh = plsc.VectorSubcoreMesh(core_axis_name="c", subcore_axis_name="s")

_sc_call = pl.kernel(
    _sc_body,
    out_type=(
        jax.ShapeDtypeStruct((NC * NS * BL * D,), jnp.float32),
        jax.ShapeDtypeStruct((NC * NS * BL * D,), jnp.int32),
    ),
    mesh=_mesh,
    scratch_types=[
        pltpu.VMEM((16,), jnp.int32),
        pltpu.VMEM((RB, D), jnp.float32),
        pltpu.VMEM((RB, D), jnp.float32),
        pltpu.VMEM((BL * D,), jnp.float32),
        pltpu.VMEM((BL * D,), jnp.int32),
        pltpu.SemaphoreType.DMA,
        pltpu.SemaphoreType.DMA,
    ],
    compiler_params=pltpu.CompilerParams(needs_layout_passes=False),
)


def _tc_combine(pv_ref, pi_ref, outv_ref, outi_ref):
    for c in range(NC):
        m = pv_ref[c * NS]
        ii = pi_ref[c * NS]
        for j in range(1, NS):
            x = pv_ref[c * NS + j]
            ix = pi_ref[c * NS + j]
            take = (x > m) | ((x == m) & (ix < ii))
            m = jnp.where(take, x, m)
            ii = jnp.where(take, ix, ii)
        col = lax.broadcasted_iota(jnp.int32, (BL, D), 1)
        outv_ref[c * BL:(c + 1) * BL, :] = m
        outi_ref[c * BL:(c + 1) * BL, :] = ii * D + col


_tc_call = pl.pallas_call(
    _tc_combine,
    out_shape=(
        jax.ShapeDtypeStruct((B, D), jnp.float32),
        jax.ShapeDtypeStruct((B, D), jnp.int32),
    ),
)


@jax.jit
def kernel(hidden, child_counts):
    pv, pi = _sc_call(hidden, child_counts)
    pooled, idx = _tc_call(pv.reshape(NC * NS, BL, D),
                           pi.reshape(NC * NS, BL, D))
    return pooled.reshape(B, 1, D), idx.reshape(B, 1, 1, D)


# speedup vs baseline: 3.5707x; 1.0797x over previous
import jax
import jax.numpy as jnp
from jax import lax
from jax.experimental import pallas as pl
from jax.experimental.pallas import tpu as pltpu
from jax.experimental.pallas import tpu_sc as plsc

B = 16
N = 2048
D = 1024
NC = 2
NS = 16
BL = B // NC
RB = 48
NV = D // 16
FB = 8
NEG = -99999.0


def _sc_body(hidden, counts, pv, pi, counts_v, buf0, buf1, acc_v, acc_i,
             sem0, sem1):
    c = lax.axis_index("c")
    s = lax.axis_index("s")

    pltpu.sync_copy(counts, counts_v)
    lane = lax.broadcasted_iota(jnp.int32, (16,), 0)
    eff_all = jnp.minimum(jnp.maximum(counts_v[...], 1), N)

    negv = jnp.full((16,), NEG, jnp.float32)
    zerov = jnp.zeros((16,), jnp.int32)

    def init_b(bl, _):
        for v in range(NV):
            acc_v[pl.ds(bl * D + v * 16, 16)] = negv
            acc_i[pl.ds(bl * D + v * 16, 16)] = zerov
        return 0
    lax.fori_loop(0, BL, init_b, 0)

    effs, nchs = [], []
    for t in range(BL):
        e = jnp.max(jnp.where(lane == c * BL + t, eff_all, 0))
        effs.append(e)
        nchs.append((e + (RB - 1)) // RB)
    pref = [jnp.int32(0)]
    for t in range(BL):
        pref.append(pref[t] + nchs[t])
    total = pref[BL]
    mine = jnp.maximum((total - s + (NS - 1)) // NS, 0)

    def locate(j):
        b_l = jnp.int32(0)
        base = jnp.int32(0)
        e_s = effs[0]
        for t in range(1, BL):
            cond = j >= pref[t]
            b_l = jnp.where(cond, t, b_l)
            base = jnp.where(cond, pref[t], base)
            e_s = jnp.where(cond, effs[t], e_s)
        return b_l, j - base, e_s

    def dma(b_l, k, buf, sem):
        return pltpu.make_async_copy(
            hidden.at[c * BL + b_l, pl.ds(k * RB, RB), :], buf, sem)

    def compute_item(b_l, k, e_s, buf):
        cnt = jnp.minimum(RB, e_s - k * RB)
        npair = cnt // 2
        for fb in range(FB):
            m = [acc_v[pl.ds(b_l * D + fb * 128 + v * 16, 16)]
                 for v in range(8)]
            ii = [acc_i[pl.ds(b_l * D + fb * 128 + v * 16, 16)]
                  for v in range(8)]
            g = jnp.full((16,), k * RB, jnp.int32)

            def pair_body(r2, carry):
                mm = list(carry[0:8])
                jj = list(carry[8:16])
                gg = carry[16]
                gg1 = gg + 1
                r = 2 * r2
                for v in range(8):
                    xa = buf[r, pl.ds(fb * 128 + v * 16, 16)]
                    xb = buf[r + 1, pl.ds(fb * 128 + v * 16, 16)]
                    pm = jnp.maximum(xa, xb)
                    pidx = jnp.where(xb > xa, gg1, gg)
                    cge = pm > mm[v]
                    mm[v] = jnp.maximum(mm[v], pm)
                    jj[v] = jnp.where(cge, pidx, jj[v])
                return tuple(mm) + tuple(jj) + (gg + 2,)

            out = lax.fori_loop(0, npair, pair_body,
                                tuple(m) + tuple(ii) + (g,))
            for v in range(8):
                acc_v[pl.ds(b_l * D + fb * 128 + v * 16, 16)] = out[v]
                acc_i[pl.ds(b_l * D + fb * 128 + v * 16, 16)] = out[8 + v]

        @pl.when(cnt % 2 == 1)
        def _():
            gt = jnp.full((16,), k * RB + cnt - 1, jnp.int32)
            for v in range(NV):
                x = buf[cnt - 1, pl.ds(v * 16, 16)]
                mv = acc_v[pl.ds(b_l * D + v * 16, 16)]
                iv = acc_i[pl.ds(b_l * D + v * 16, 16)]
                cge = x > mv
                acc_v[pl.ds(b_l * D + v * 16, 16)] = jnp.where(cge, x, mv)
                acc_i[pl.ds(b_l * D + v * 16, 16)] = jnp.where(cge, gt, iv)

    b0, k0, e0 = locate(s)

    @pl.when(mine > 0)
    def _():
        dma(b0, k0, buf0, sem0).start()

    bufs = (buf0, buf1)
    sems = (sem0, sem1)

    def pair_body(p, _):
        for q in (0, 1):
            item = 2 * p + q

            @pl.when(item < mine)
            def _(item=item, q=q):
                j = s + NS * item
                b_l, k, e_s = locate(j)
                dma(b_l, k, bufs[q], sems[q]).wait()
                nitem = item + 1

                @pl.when(nitem < mine)
                def _():
                    nb, nk, _ne = locate(s + NS * nitem)
                    dma(nb, nk, bufs[1 - q], sems[1 - q]).start()

                compute_item(b_l, k, e_s, bufs[q])
        return 0

    lax.fori_loop(0, (mine + 1) // 2, pair_body, 0)

    w = c * NS + s
    for bl in range(BL):
        pltpu.sync_copy(acc_v.at[pl.ds(bl * D, D)],
                        pv.at[pl.ds((w * BL + bl) * D, D)])
        pltpu.sync_copy(acc_i.at[pl.ds(bl * D, D)],
                        pi.at[pl.ds((w * BL + bl) * D, D)])


_mesh = plsc.VectorSubcoreMesh(core_axis_name="c", subcore_axis_name="s")

_sc_call = pl.kernel(
    _sc_body,
    out_type=(
        jax.ShapeDtypeStruct((NC * NS * BL * D,), jnp.float32),
        jax.ShapeDtypeStruct((NC * NS * BL * D,), jnp.int32),
    ),
    mesh=_mesh,
    scratch_types=[
        pltpu.VMEM((16,), jnp.int32),
        pltpu.VMEM((RB, D), jnp.float32),
        pltpu.VMEM((RB, D), jnp.float32),
        pltpu.VMEM((BL * D,), jnp.float32),
        pltpu.VMEM((BL * D,), jnp.int32),
        pltpu.SemaphoreType.DMA,
        pltpu.SemaphoreType.DMA,
    ],
    compiler_params=pltpu.CompilerParams(needs_layout_passes=False),
)


def _tc_combine(pv_ref, pi_ref, outv_ref, outi_ref):
    col = lax.broadcasted_iota(jnp.int32, (D,), 0)
    for bg in range(B):
        c, bl = bg // BL, bg % BL
        m = pv_ref[pl.ds((c * NS * BL + bl) * D, D)]
        ii = pi_ref[pl.ds((c * NS * BL + bl) * D, D)]
        for j in range(1, NS):
            off = (((c * NS) + j) * BL + bl) * D
            x = pv_ref[pl.ds(off, D)]
            ix = pi_ref[pl.ds(off, D)]
            take = (x > m) | ((x == m) & (ix < ii))
            m = jnp.where(take, x, m)
            ii = jnp.where(take, ix, ii)
        outv_ref[bg, 0, :] = m
        outi_ref[bg, 0, 0, :] = ii * D + col


_tc_call = pl.pallas_call(
    _tc_combine,
    out_shape=(
        jax.ShapeDtypeStruct((B, 1, D), jnp.float32),
        jax.ShapeDtypeStruct((B, 1, 1, D), jnp.int32),
    ),
)


@jax.jit
def kernel(hidden, child_counts):
    pv, pi = _sc_call(hidden, child_counts)
    return _tc_call(pv, pi)
